# baseline (speedup 1.0000x reference)
import functools

import jax
import jax.numpy as jnp
from jax import lax
from jax.experimental import pallas as pl
from jax.experimental.pallas import tpu as pltpu
from jax.experimental.pallas import tpu_sc as plsc


def _grouping_body(points_hbm, idx_hbm, out_hbm, idx_v, row_v, out_v):
    B, C, N = points_hbm.shape
    _, S, Q = idx_hbm.shape
    info = plsc.get_sparse_core_info()
    NC, NS, L = info.num_cores, info.num_subcores, info.num_lanes
    NW = NC * NS
    tiles_per_b = NW // B
    c_per_tile = C // tiles_per_b

    wid = lax.axis_index("s") * NC + lax.axis_index("c")
    b = wid // tiles_per_b
    c0 = (wid % tiles_per_b) * c_per_tile

    pltpu.sync_copy(idx_hbm.at[b], idx_v)

    U = 8

    def q_step(k, s):
        base = k * (L * U)
        for u in range(U):
            off = base + u * L
            iv = idx_v[s, pl.ds(off, L)]
            out_v[s, pl.ds(off, L)] = plsc.load_gather(row_v, [iv])
        return s

    def s_step(s, _):
        lax.fori_loop(0, Q // (L * U), q_step, s)
        return 0

    for j in range(c_per_tile):
        cc = c0 + j
        pltpu.sync_copy(points_hbm.at[b, cc], row_v)
        lax.fori_loop(0, S, s_step, 0)
        pltpu.sync_copy(out_v, out_hbm.at[b, cc])


def _make_grouping(B, C, N, S, Q):
    return functools.partial(
        pl.kernel,
        out_type=jax.ShapeDtypeStruct((B, C, S, Q), jnp.float32),
        mesh=plsc.VectorSubcoreMesh(core_axis_name="c", subcore_axis_name="s"),
        compiler_params=pltpu.CompilerParams(needs_layout_passes=False),
        scratch_types=[
            pltpu.VMEM((S, Q), jnp.int32),
            pltpu.VMEM((N,), jnp.float32),
            pltpu.VMEM((S, Q), jnp.float32),
        ],
    )(_grouping_body)


@jax.jit
def kernel(points, idx):
    B, C, N = points.shape
    _, npoint, nsample = idx.shape
    idx_t = jnp.transpose(idx.astype(jnp.int32), (0, 2, 1))
    out_t = _make_grouping(B, C, N, nsample, npoint)(points, idx_t)
    return jnp.transpose(out_t, (0, 1, 3, 2))


# speedup vs baseline: 3.2060x; 3.2060x over previous
import functools

import jax
import jax.numpy as jnp
from jax import lax
from jax.experimental import pallas as pl
from jax.experimental.pallas import tpu as pltpu
from jax.experimental.pallas import tpu_sc as plsc


def _grouping_body(points_hbm, idx_hbm, out_hbm, idx_v, row_v, out_v):
    B, C, N = points_hbm.shape
    _, S, Q = idx_hbm.shape
    info = plsc.get_sparse_core_info()
    NC, NS, L = info.num_cores, info.num_subcores, info.num_lanes
    NW = NC * NS
    tiles_per_b = NW // B
    c_per_tile = C // tiles_per_b

    wid = lax.axis_index("s") * NC + lax.axis_index("c")
    b = wid // tiles_per_b
    c0 = (wid % tiles_per_b) * c_per_tile

    QS = S * Q
    blocks_per_s = Q // L
    pltpu.sync_copy(idx_hbm.at[b], idx_v)

    for j in range(c_per_tile):
        cc = c0 + j
        pltpu.sync_copy(points_hbm.at[b, cc], row_v)

        @plsc.parallel_loop(0, QS // L, unroll=8)
        def _gather(i):
            s = i // blocks_per_s
            qo = (i % blocks_per_s) * L
            iv = idx_v[s, pl.ds(qo, L)]
            out_v[s, pl.ds(qo, L)] = plsc.load_gather(row_v, [iv])

        pltpu.sync_copy(out_v, out_hbm.at[b, cc])


def _make_grouping(B, C, N, S, Q):
    return functools.partial(
        pl.kernel,
        out_type=jax.ShapeDtypeStruct((B, C, S, Q), jnp.float32),
        mesh=plsc.VectorSubcoreMesh(core_axis_name="c", subcore_axis_name="s"),
        compiler_params=pltpu.CompilerParams(needs_layout_passes=False),
        scratch_types=[
            pltpu.VMEM((S, Q), jnp.int32),
            pltpu.VMEM((N,), jnp.float32),
            pltpu.VMEM((S, Q), jnp.float32),
        ],
    )(_grouping_body)


@jax.jit
def kernel(points, idx):
    B, C, N = points.shape
    _, npoint, nsample = idx.shape
    idx_t = jnp.transpose(idx.astype(jnp.int32), (0, 2, 1))
    out_t = _make_grouping(B, C, N, nsample, npoint)(points, idx_t)
    return jnp.transpose(out_t, (0, 1, 3, 2))
